# initial kernel scaffold (unmeasured)
import jax
import jax.numpy as jnp
from jax import lax
from jax.experimental import pallas as pl
from jax.experimental.pallas import tpu as pltpu

N_DEV = 32
B, SQ, D = 4, 256, 1024
HQ_LOCAL, DH = 8, 128
ROWS = B * SQ
CHUNK = ROWS // N_DEV
SCALE = 0.08838834764831843


def _body(x_ref, wq_ref, wo_ref, wk_ref, wv_ref, out_ref,
          attn_ref, comm_ref, send_sems, recv_sems):
    i = lax.axis_index("i")
    left = lax.rem(i + N_DEV - 1, N_DEV)
    right = lax.rem(i + 1, N_DEV)

    barrier_sem = pltpu.get_barrier_semaphore()
    for nbr in (left, right):
        pl.semaphore_signal(
            barrier_sem, inc=1,
            device_id=(nbr,), device_id_type=pl.DeviceIdType.MESH,
        )
    pl.semaphore_wait(barrier_sem, 2)

    for b in range(B):
        xb = x_ref[b * SQ:(b + 1) * SQ, :]
        qb = jnp.dot(xb, wq_ref[...], preferred_element_type=jnp.float32)
        kb = jnp.dot(xb, wk_ref[...], preferred_element_type=jnp.float32)
        vb = jnp.dot(xb, wv_ref[...], preferred_element_type=jnp.float32)
        for h in range(HQ_LOCAL):
            g = h // 4
            q = qb[:, h * DH:(h + 1) * DH]
            k = kb[:, g * DH:(g + 1) * DH]
            v = vb[:, g * DH:(g + 1) * DH]
            s = lax.dot_general(
                q, k, (((1,), (1,)), ((), ())),
                preferred_element_type=jnp.float32,
            ) * SCALE
            m = jnp.max(s, axis=1, keepdims=True)
            p = jnp.exp(s - m)
            l = jnp.sum(p, axis=1, keepdims=True)
            o = jnp.dot(p, v, preferred_element_type=jnp.float32) / l
            attn_ref[:, h * DH:(h + 1) * DH] = o
        out_ref[b * SQ:(b + 1) * SQ, :] = jnp.dot(
            attn_ref[...], wo_ref[...], preferred_element_type=jnp.float32
        )

    for s in range(N_DEV - 1):
        c_send = (i + 2 * N_DEV - s) % N_DEV
        rdma = pltpu.make_async_remote_copy(
            src_ref=out_ref.at[pl.ds(c_send * CHUNK, CHUNK), :],
            dst_ref=comm_ref.at[s],
            send_sem=send_sems.at[s],
            recv_sem=recv_sems.at[s],
            device_id=(right,),
            device_id_type=pl.DeviceIdType.MESH,
        )
        rdma.start()
        rdma.wait()
        c_recv = (i + 2 * N_DEV - 1 - s) % N_DEV
        r = c_recv * CHUNK
        out_ref[pl.ds(r, CHUNK), :] = (
            out_ref[pl.ds(r, CHUNK), :] + comm_ref[s]
        )

    for s in range(N_DEV - 1):
        c = (i + 2 * N_DEV + 1 - s) % N_DEV
        r = c * CHUNK
        rdma = pltpu.make_async_remote_copy(
            src_ref=out_ref.at[pl.ds(r, CHUNK), :],
            dst_ref=out_ref.at[pl.ds(r, CHUNK), :],
            send_sem=send_sems.at[N_DEV - 1 + s],
            recv_sem=recv_sems.at[N_DEV - 1 + s],
            device_id=(right,),
            device_id_type=pl.DeviceIdType.MESH,
        )
        rdma.start()
        rdma.wait()


def kernel(x, Wq, Wo, Wk, Wv):
    i = lax.axis_index("i")
    xf = x.reshape(ROWS, D)
    wk_sl = lax.dynamic_slice(Wk, (0, i * 2 * DH), (D, 2 * DH))
    wv_sl = lax.dynamic_slice(Wv, (0, i * 2 * DH), (D, 2 * DH))

    out = pl.pallas_call(
        _body,
        out_shape=jax.ShapeDtypeStruct((ROWS, D), jnp.float32),
        in_specs=[pl.BlockSpec(memory_space=pltpu.VMEM)] * 5,
        out_specs=pl.BlockSpec(memory_space=pltpu.VMEM),
        scratch_shapes=[
            pltpu.VMEM((SQ, D), jnp.float32),
            pltpu.VMEM((N_DEV - 1, CHUNK, D), jnp.float32),
            pltpu.SemaphoreType.DMA((2 * (N_DEV - 1),)),
            pltpu.SemaphoreType.DMA((2 * (N_DEV - 1),)),
        ],
        compiler_params=pltpu.CompilerParams(collective_id=0),
    )(xf, Wq, Wo, wk_sl, wv_sl)
    return out.reshape(B, SQ, D)


# baseline (device time: 225643 ns/iter reference)
import os

import jax
import jax.numpy as jnp
from jax import lax
from jax.experimental import pallas as pl
from jax.experimental.pallas import tpu as pltpu

_STAGE = os.environ.get("KERNEL_STAGE", "full")

N_DEV = 32
B, SQ, D = 4, 256, 1024
HQ_LOCAL, DH = 8, 128
ROWS = B * SQ
CHUNK = ROWS // N_DEV
SCALE = 0.08838834764831843


def _body(x_ref, wq_ref, wo_ref, wk_ref, wv_ref, out_ref,
          attn_ref, comm_ref, send_sems, recv_sems):
    i = lax.axis_index("i")
    left = lax.rem(i + N_DEV - 1, N_DEV)
    right = lax.rem(i + 1, N_DEV)

    if _STAGE != "compute":
        barrier_sem = pltpu.get_barrier_semaphore()
        for nbr in (left, right):
            pl.semaphore_signal(
                barrier_sem, inc=1,
                device_id=(nbr,), device_id_type=pl.DeviceIdType.MESH,
            )
        pl.semaphore_wait(barrier_sem, 2)

    for b in range(B):
        xb = x_ref[b * SQ:(b + 1) * SQ, :]
        qb = jnp.dot(xb, wq_ref[...], preferred_element_type=jnp.float32)
        kb = jnp.dot(xb, wk_ref[...], preferred_element_type=jnp.float32)
        vb = jnp.dot(xb, wv_ref[...], preferred_element_type=jnp.float32)
        for h in range(HQ_LOCAL):
            g = h // 4
            q = qb[:, h * DH:(h + 1) * DH]
            k = kb[:, g * DH:(g + 1) * DH]
            v = vb[:, g * DH:(g + 1) * DH]
            s = lax.dot_general(
                q, k, (((1,), (1,)), ((), ())),
                preferred_element_type=jnp.float32,
            ) * SCALE
            m = jnp.max(s, axis=1, keepdims=True)
            p = jnp.exp(s - m)
            l = jnp.sum(p, axis=1, keepdims=True)
            o = jnp.dot(p, v, preferred_element_type=jnp.float32) / l
            attn_ref[:, h * DH:(h + 1) * DH] = o
        out_ref[b * SQ:(b + 1) * SQ, :] = jnp.dot(
            attn_ref[...], wo_ref[...], preferred_element_type=jnp.float32
        )

    if _STAGE == "compute":
        return

    for s in range(N_DEV - 1):
        c_send = (i + 2 * N_DEV - s) % N_DEV
        rdma = pltpu.make_async_remote_copy(
            src_ref=out_ref.at[pl.ds(c_send * CHUNK, CHUNK), :],
            dst_ref=comm_ref.at[s],
            send_sem=send_sems.at[s],
            recv_sem=recv_sems.at[s],
            device_id=(right,),
            device_id_type=pl.DeviceIdType.MESH,
        )
        rdma.start()
        rdma.wait()
        c_recv = (i + 2 * N_DEV - 1 - s) % N_DEV
        r = c_recv * CHUNK
        out_ref[pl.ds(r, CHUNK), :] = (
            out_ref[pl.ds(r, CHUNK), :] + comm_ref[s]
        )

    if _STAGE == "rs":
        return

    for s in range(N_DEV - 1):
        c = (i + 2 * N_DEV + 1 - s) % N_DEV
        r = c * CHUNK
        rdma = pltpu.make_async_remote_copy(
            src_ref=out_ref.at[pl.ds(r, CHUNK), :],
            dst_ref=out_ref.at[pl.ds(r, CHUNK), :],
            send_sem=send_sems.at[N_DEV - 1 + s],
            recv_sem=recv_sems.at[N_DEV - 1 + s],
            device_id=(right,),
            device_id_type=pl.DeviceIdType.MESH,
        )
        rdma.start()
        rdma.wait()


def kernel(x, Wq, Wo, Wk, Wv):
    i = lax.axis_index("i")
    xf = x.reshape(ROWS, D)
    wk_sl = lax.dynamic_slice(Wk, (0, i * 2 * DH), (D, 2 * DH))
    wv_sl = lax.dynamic_slice(Wv, (0, i * 2 * DH), (D, 2 * DH))

    out = pl.pallas_call(
        _body,
        out_shape=jax.ShapeDtypeStruct((ROWS, D), jnp.float32),
        in_specs=[pl.BlockSpec(memory_space=pltpu.VMEM)] * 5,
        out_specs=pl.BlockSpec(memory_space=pltpu.VMEM),
        scratch_shapes=[
            pltpu.VMEM((SQ, D), jnp.float32),
            pltpu.VMEM((N_DEV - 1, CHUNK, D), jnp.float32),
            pltpu.SemaphoreType.DMA((2 * (N_DEV - 1),)),
            pltpu.SemaphoreType.DMA((2 * (N_DEV - 1),)),
        ],
        compiler_params=(
            None if _STAGE == "compute"
            else pltpu.CompilerParams(collective_id=0)
        ),
    )(xf, Wq, Wo, wk_sl, wv_sl)
    return out.reshape(B, SQ, D)


# device time: 138551 ns/iter; 1.6286x vs baseline; 1.6286x over previous
import os

import jax
import jax.numpy as jnp
from jax import lax
from jax.experimental import pallas as pl
from jax.experimental.pallas import tpu as pltpu

_STAGE = os.environ.get("KERNEL_STAGE", "full")

N_DEV = 32
B, SQ, D = 4, 256, 1024
HQ_LOCAL, DH = 8, 128
ROWS = B * SQ
HALF = ROWS // 2
QTR = HALF // 4
CH = QTR // 4
SCALE = 0.08838834764831843


def _ring_idx(x, y, z):
    return 8 * z + 2 * y + jnp.where(lax.rem(y, 2) == 0, x, 1 - x)


def _body(x_ref, wq_ref, wo_ref, wk_ref, wv_ref, out_ref,
          attn_ref, comm_x, comm_y, comm_z, send_sems, recv_sems):
    i = lax.axis_index("i")
    z = i // 8
    r = lax.rem(i, 8)
    y = r // 2
    x = jnp.where(lax.rem(y, 2) == 0, lax.rem(r, 2), 1 - lax.rem(r, 2))

    x_partner = _ring_idx(1 - x, y, z)
    y_next = _ring_idx(x, lax.rem(y + 1, 4), z)
    y_prev = _ring_idx(x, lax.rem(y + 3, 4), z)
    z_next = _ring_idx(x, y, lax.rem(z + 1, 4))
    z_prev = _ring_idx(x, y, lax.rem(z + 3, 4))

    for b in range(B):
        xb = x_ref[b * SQ:(b + 1) * SQ, :]
        qb = jnp.dot(xb, wq_ref[...], preferred_element_type=jnp.float32)
        kb = jnp.dot(xb, wk_ref[...], preferred_element_type=jnp.float32)
        vb = jnp.dot(xb, wv_ref[...], preferred_element_type=jnp.float32)
        for h in range(HQ_LOCAL):
            g = h // 4
            q = qb[:, h * DH:(h + 1) * DH]
            k = kb[:, g * DH:(g + 1) * DH]
            v = vb[:, g * DH:(g + 1) * DH]
            s = lax.dot_general(
                q, k, (((1,), (1,)), ((), ())),
                preferred_element_type=jnp.float32,
            ) * SCALE
            m = jnp.max(s, axis=1, keepdims=True)
            p = jnp.exp(s - m)
            l = jnp.sum(p, axis=1, keepdims=True)
            o = jnp.dot(p, v, preferred_element_type=jnp.float32) / l
            attn_ref[:, h * DH:(h + 1) * DH] = o
        out_ref[b * SQ:(b + 1) * SQ, :] = jnp.dot(
            attn_ref[...], wo_ref[...], preferred_element_type=jnp.float32
        )

    if _STAGE == "compute":
        return

    barrier_sem = pltpu.get_barrier_semaphore()
    for nbr in (x_partner, y_next, y_prev, z_next, z_prev):
        pl.semaphore_signal(
            barrier_sem, inc=1,
            device_id=(nbr,), device_id_type=pl.DeviceIdType.MESH,
        )
    pl.semaphore_wait(barrier_sem, 5)

    step = 0

    def _xfer(src, dst, target, sem_idx):
        rdma = pltpu.make_async_remote_copy(
            src_ref=src, dst_ref=dst,
            send_sem=send_sems.at[sem_idx], recv_sem=recv_sems.at[sem_idx],
            device_id=(target,), device_id_type=pl.DeviceIdType.MESH,
        )
        rdma.start()
        rdma.wait()

    _xfer(out_ref.at[pl.ds((1 - x) * HALF, HALF), :], comm_x, x_partner, step)
    step += 1
    mh = x * HALF
    out_ref[pl.ds(mh, HALF), :] = out_ref[pl.ds(mh, HALF), :] + comm_x[...]

    for s in range(3):
        q_send = lax.rem(y + 7 - s, 4)
        _xfer(out_ref.at[pl.ds(mh + q_send * QTR, QTR), :],
              comm_y.at[s], y_next, step)
        step += 1
        q_recv = lax.rem(y + 6 - s, 4)
        rr = mh + q_recv * QTR
        out_ref[pl.ds(rr, QTR), :] = out_ref[pl.ds(rr, QTR), :] + comm_y[s]

    mq = mh + y * QTR

    for s in range(3):
        c_send = lax.rem(z + 7 - s, 4)
        _xfer(out_ref.at[pl.ds(mq + c_send * CH, CH), :],
              comm_z.at[s], z_next, step)
        step += 1
        c_recv = lax.rem(z + 6 - s, 4)
        rr = mq + c_recv * CH
        out_ref[pl.ds(rr, CH), :] = out_ref[pl.ds(rr, CH), :] + comm_z[s]

    for s in range(3):
        c = lax.rem(z + 4 - s, 4)
        sl = out_ref.at[pl.ds(mq + c * CH, CH), :]
        _xfer(sl, sl, z_next, step)
        step += 1

    for s in range(3):
        q = lax.rem(y + 4 - s, 4)
        sl = out_ref.at[pl.ds(mh + q * QTR, QTR), :]
        _xfer(sl, sl, y_next, step)
        step += 1

    sl = out_ref.at[pl.ds(mh, HALF), :]
    _xfer(sl, sl, x_partner, step)


def kernel(x, Wq, Wo, Wk, Wv):
    i = lax.axis_index("i")
    xf = x.reshape(ROWS, D)
    wk_sl = lax.dynamic_slice(Wk, (0, i * 2 * DH), (D, 2 * DH))
    wv_sl = lax.dynamic_slice(Wv, (0, i * 2 * DH), (D, 2 * DH))

    out = pl.pallas_call(
        _body,
        out_shape=jax.ShapeDtypeStruct((ROWS, D), jnp.float32),
        in_specs=[pl.BlockSpec(memory_space=pltpu.VMEM)] * 5,
        out_specs=pl.BlockSpec(memory_space=pltpu.VMEM),
        scratch_shapes=[
            pltpu.VMEM((SQ, D), jnp.float32),
            pltpu.VMEM((HALF, D), jnp.float32),
            pltpu.VMEM((3, QTR, D), jnp.float32),
            pltpu.VMEM((3, CH, D), jnp.float32),
            pltpu.SemaphoreType.DMA((14,)),
            pltpu.SemaphoreType.DMA((14,)),
        ],
        compiler_params=(
            None if _STAGE == "compute"
            else pltpu.CompilerParams(collective_id=0)
        ),
    )(xf, Wq, Wo, wk_sl, wv_sl)
    return out.reshape(B, SQ, D)


# device time: 107119 ns/iter; 2.1065x vs baseline; 1.2934x over previous
import os

import jax
import jax.numpy as jnp
from jax import lax
from jax.experimental import pallas as pl
from jax.experimental.pallas import tpu as pltpu

_STAGE = os.environ.get("KERNEL_STAGE", "full")

N_DEV = 32
B, SQ, D = 4, 256, 1024
HQ_LOCAL, DH = 8, 128
ROWS = B * SQ
SCALE = 0.08838834764831843

AH, AQ, AC = 256, 64, 16
BZ, BY, BX = 128, 32, 16
B0 = 512


def _ring_idx(x, y, z):
    return 8 * z + 2 * y + jnp.where(lax.rem(y, 2) == 0, x, 1 - x)


def _compute_batch(b, x_ref, wq_ref, wo_ref, wk_ref, wv_ref, attn_ref, out_ref):
    xb = x_ref[b * SQ:(b + 1) * SQ, :]
    qb = jnp.dot(xb, wq_ref[...], preferred_element_type=jnp.float32)
    kb = jnp.dot(xb, wk_ref[...], preferred_element_type=jnp.float32)
    vb = jnp.dot(xb, wv_ref[...], preferred_element_type=jnp.float32)
    for h in range(HQ_LOCAL):
        g = h // 4
        q = qb[:, h * DH:(h + 1) * DH]
        k = kb[:, g * DH:(g + 1) * DH]
        v = vb[:, g * DH:(g + 1) * DH]
        s = lax.dot_general(
            q, k, (((1,), (1,)), ((), ())),
            preferred_element_type=jnp.float32,
        ) * SCALE
        m = jnp.max(s, axis=1, keepdims=True)
        p = jnp.exp(s - m)
        l = jnp.sum(p, axis=1, keepdims=True)
        o = jnp.dot(p, v, preferred_element_type=jnp.float32) / l
        attn_ref[:, h * DH:(h + 1) * DH] = o
    out_ref[b * SQ:(b + 1) * SQ, :] = jnp.dot(
        attn_ref[...], wo_ref[...], preferred_element_type=jnp.float32
    )


def _body(x_ref, wq_ref, wo_ref, wk_ref, wv_ref, out_ref,
          attn_ref, comm_ax, comm_ay, comm_az, comm_bz, comm_by, comm_bx,
          send_sems, recv_sems):
    i = lax.axis_index("i")
    z = i // 8
    r = lax.rem(i, 8)
    y = r // 2
    x = jnp.where(lax.rem(y, 2) == 0, lax.rem(r, 2), 1 - lax.rem(r, 2))

    x_partner = _ring_idx(1 - x, y, z)
    y_next = _ring_idx(x, lax.rem(y + 1, 4), z)
    y_prev = _ring_idx(x, lax.rem(y + 3, 4), z)
    z_next = _ring_idx(x, y, lax.rem(z + 1, 4))
    z_prev = _ring_idx(x, y, lax.rem(z + 3, 4))

    if _STAGE == "compute":
        for b in range(B):
            _compute_batch(b, x_ref, wq_ref, wo_ref, wk_ref, wv_ref,
                           attn_ref, out_ref)
        return

    barrier_sem = pltpu.get_barrier_semaphore()
    for nbr in (x_partner, y_next, y_prev, z_next, z_prev):
        pl.semaphore_signal(
            barrier_sem, inc=1,
            device_id=(nbr,), device_id_type=pl.DeviceIdType.MESH,
        )
    pl.semaphore_wait(barrier_sem, 5)

    sends = []

    def _start(src, dst, target, k):
        rdma = pltpu.make_async_remote_copy(
            src_ref=src, dst_ref=dst,
            send_sem=send_sems.at[k], recv_sem=recv_sems.at[k],
            device_id=(target,), device_id_type=pl.DeviceIdType.MESH,
        )
        rdma.start()
        sends.append(rdma)
        return rdma

    mhA = x * AH
    mqA = mhA + y * AQ
    mzB = B0 + z * BZ
    myB = mzB + y * BY

    _compute_batch(0, x_ref, wq_ref, wo_ref, wk_ref, wv_ref, attn_ref, out_ref)
    _compute_batch(1, x_ref, wq_ref, wo_ref, wk_ref, wv_ref, attn_ref, out_ref)
    ra = _start(out_ref.at[pl.ds((1 - x) * AH, AH), :], comm_ax, x_partner, 0)
    _compute_batch(2, x_ref, wq_ref, wo_ref, wk_ref, wv_ref, attn_ref, out_ref)
    _compute_batch(3, x_ref, wq_ref, wo_ref, wk_ref, wv_ref, attn_ref, out_ref)
    rb = _start(out_ref.at[pl.ds(B0 + lax.rem(z + 1, 4) * BZ, BZ), :],
                comm_bz.at[0], z_prev, 1)
    ra.wait_recv()
    out_ref[pl.ds(mhA, AH), :] = out_ref[pl.ds(mhA, AH), :] + comm_ax[...]
    rb.wait_recv()
    rr = B0 + lax.rem(z + 2, 4) * BZ
    out_ref[pl.ds(rr, BZ), :] = out_ref[pl.ds(rr, BZ), :] + comm_bz[0]

    a_rs = (
        [("y", s) for s in range(3)] + [("z", s) for s in range(3)]
    )
    b_rs = (
        [("z", s) for s in range(1, 3)] + [("y", s) for s in range(3)] + [("x", 0)]
    )
    for slot in range(6):
        k = 2 + 2 * slot
        lvl, s = a_rs[slot]
        if lvl == "y":
            a_src = mhA + lax.rem(y + 7 - s, 4) * AQ
            ra = _start(out_ref.at[pl.ds(a_src, AQ), :], comm_ay.at[s],
                        y_next, k)
            a_acc, a_len = mhA + lax.rem(y + 6 - s, 4) * AQ, AQ
            a_load = lambda s=s: comm_ay[s]
        else:
            a_src = mqA + lax.rem(z + 7 - s, 4) * AC
            ra = _start(out_ref.at[pl.ds(a_src, AC), :], comm_az.at[s],
                        z_next, k)
            a_acc, a_len = mqA + lax.rem(z + 6 - s, 4) * AC, AC
            a_load = lambda s=s: comm_az[s]
        lvl, s = b_rs[slot]
        if lvl == "z":
            b_src = B0 + lax.rem(z + 1 + s, 4) * BZ
            rb = _start(out_ref.at[pl.ds(b_src, BZ), :], comm_bz.at[s],
                        z_prev, k + 1)
            b_acc, b_len = B0 + lax.rem(z + 2 + s, 4) * BZ, BZ
            b_load = lambda s=s: comm_bz[s]
        elif lvl == "y":
            b_src = mzB + lax.rem(y + 1 + s, 4) * BY
            rb = _start(out_ref.at[pl.ds(b_src, BY), :], comm_by.at[s],
                        y_prev, k + 1)
            b_acc, b_len = mzB + lax.rem(y + 2 + s, 4) * BY, BY
            b_load = lambda s=s: comm_by[s]
        else:
            rb = _start(out_ref.at[pl.ds(myB + (1 - x) * BX, BX), :],
                        comm_bx, x_partner, k + 1)
            b_acc, b_len = myB + x * BX, BX
            b_load = lambda: comm_bx[...]
        ra.wait_recv()
        out_ref[pl.ds(a_acc, a_len), :] = (
            out_ref[pl.ds(a_acc, a_len), :] + a_load()
        )
        rb.wait_recv()
        out_ref[pl.ds(b_acc, b_len), :] = (
            out_ref[pl.ds(b_acc, b_len), :] + b_load()
        )

    a_ag = [("z", s) for s in range(3)] + [("y", s) for s in range(3)] + [("x", 0)]
    b_ag = [("x", 0)] + [("y", s) for s in range(3)] + [("z", s) for s in range(3)]
    for slot in range(7):
        k = 14 + 2 * slot
        lvl, s = a_ag[slot]
        if lvl == "z":
            off, ln, tgt = mqA + lax.rem(z + 4 - s, 4) * AC, AC, z_next
        elif lvl == "y":
            off, ln, tgt = mhA + lax.rem(y + 4 - s, 4) * AQ, AQ, y_next
        else:
            off, ln, tgt = mhA, AH, x_partner
        sl = out_ref.at[pl.ds(off, ln), :]
        ra = _start(sl, sl, tgt, k)
        lvl, s = b_ag[slot]
        if lvl == "x":
            off, ln, tgt = myB + x * BX, BX, x_partner
        elif lvl == "y":
            off, ln, tgt = mzB + lax.rem(y + s, 4) * BY, BY, y_prev
        else:
            off, ln, tgt = B0 + lax.rem(z + s, 4) * BZ, BZ, z_prev
        sl = out_ref.at[pl.ds(off, ln), :]
        rb = _start(sl, sl, tgt, k + 1)
        ra.wait_recv()
        rb.wait_recv()

    for d in sends:
        d.wait_send()


def kernel(x, Wq, Wo, Wk, Wv):
    i = lax.axis_index("i")
    xf = x.reshape(ROWS, D)
    wk_sl = lax.dynamic_slice(Wk, (0, i * 2 * DH), (D, 2 * DH))
    wv_sl = lax.dynamic_slice(Wv, (0, i * 2 * DH), (D, 2 * DH))

    out = pl.pallas_call(
        _body,
        out_shape=jax.ShapeDtypeStruct((ROWS, D), jnp.float32),
        in_specs=[pl.BlockSpec(memory_space=pltpu.VMEM)] * 5,
        out_specs=pl.BlockSpec(memory_space=pltpu.VMEM),
        scratch_shapes=[
            pltpu.VMEM((SQ, D), jnp.float32),
            pltpu.VMEM((AH, D), jnp.float32),
            pltpu.VMEM((3, AQ, D), jnp.float32),
            pltpu.VMEM((3, AC, D), jnp.float32),
            pltpu.VMEM((3, BZ, D), jnp.float32),
            pltpu.VMEM((3, BY, D), jnp.float32),
            pltpu.VMEM((BX, D), jnp.float32),
            pltpu.SemaphoreType.DMA((28,)),
            pltpu.SemaphoreType.DMA((28,)),
        ],
        compiler_params=(
            None if _STAGE == "compute"
            else pltpu.CompilerParams(collective_id=0)
        ),
    )(xf, Wq, Wo, wk_sl, wv_sl)
    return out.reshape(B, SQ, D)
